# baseline (speedup 1.0000x reference)
import functools

import jax
import jax.numpy as jnp
from jax import lax
from jax.experimental import pallas as pl
from jax.experimental.pallas import tpu as pltpu
from jax.experimental.pallas import tpu_sc as plsc

_NC = 2
_NS = 16
_LANES = 16


def kernel(x, pe_weight):
    B, S, D = x.shape
    NW = _NC * _NS
    s_per_w = S // NW
    CH = 32
    n_chunks = s_per_w // CH
    mesh = plsc.VectorSubcoreMesh(core_axis_name="c", subcore_axis_name="s")

    @functools.partial(
        pl.kernel,
        mesh=mesh,
        out_type=jax.ShapeDtypeStruct((B, S, D), x.dtype),
        scratch_types=[
            pltpu.VMEM((CH, D), jnp.float32),
            pltpu.VMEM((CH, D), jnp.float32),
        ],
    )
    def k(x_hbm, pe_hbm, out_hbm, pe_v, x_v):
        wid = lax.axis_index("s") * _NC + lax.axis_index("c")
        s_base = wid * s_per_w
        for c in range(n_chunks):
            s0 = s_base + c * CH
            pltpu.sync_copy(pe_hbm.at[pl.ds(s0, CH), :], pe_v)
            for b in range(B):
                pltpu.sync_copy(x_hbm.at[b, pl.ds(s0, CH), :], x_v)

                def row_body(r, carry):
                    for j in range(D // _LANES):
                        sl = pl.ds(j * _LANES, _LANES)
                        plsc.addupdate(x_v.at[r, sl], pe_v[r, sl])
                    return carry

                lax.fori_loop(0, CH, row_body, 0)
                pltpu.sync_copy(x_v, out_hbm.at[b, pl.ds(s0, CH), :])

    return k(x, pe_weight)


# speedup vs baseline: 1.2135x; 1.2135x over previous
import functools

import jax
import jax.numpy as jnp
from jax import lax
from jax.experimental import pallas as pl
from jax.experimental.pallas import tpu as pltpu
from jax.experimental.pallas import tpu_sc as plsc

_NC = 2
_NS = 16
_LANES = 16


def kernel(x, pe_weight):
    B, S, D = x.shape
    NW = _NC * _NS
    s_per_w = S // NW
    CH = 32
    n_chunks = s_per_w // CH
    steps = [(c, b) for c in range(n_chunks) for b in range(B)]
    mesh = plsc.VectorSubcoreMesh(core_axis_name="c", subcore_axis_name="s")

    @functools.partial(
        pl.kernel,
        mesh=mesh,
        out_type=jax.ShapeDtypeStruct((B, S, D), x.dtype),
        scratch_types=[
            pltpu.VMEM((CH, D), jnp.float32),
            pltpu.VMEM((2, CH, D), jnp.float32),
            pltpu.SemaphoreType.DMA,
            pltpu.SemaphoreType.DMA,
            pltpu.SemaphoreType.DMA,
            pltpu.SemaphoreType.DMA,
        ],
    )
    def k(x_hbm, pe_hbm, out_hbm, pe_v, x_v, in0, in1, out0, out1):
        wid = lax.axis_index("s") * _NC + lax.axis_index("c")
        s_base = wid * s_per_w
        in_sems = (in0, in1)
        out_sems = (out0, out1)

        def s0_of(c):
            return s_base + c * CH

        c0, b0 = steps[0]
        pltpu.async_copy(x_hbm.at[b0, pl.ds(s0_of(c0), CH), :],
                         x_v.at[0], in_sems[0])
        pltpu.sync_copy(pe_hbm.at[pl.ds(s0_of(c0), CH), :], pe_v)

        for i, (c, b) in enumerate(steps):
            cur = i % 2
            nxt = (i + 1) % 2
            if i + 1 < len(steps):
                cn, bn = steps[i + 1]
                if i >= 1:
                    pltpu.make_async_copy(x_v.at[nxt],
                                          out_hbm.at[bn, pl.ds(0, CH), :],
                                          out_sems[nxt]).wait()
                pltpu.async_copy(x_hbm.at[bn, pl.ds(s0_of(cn), CH), :],
                                 x_v.at[nxt], in_sems[nxt])

            pltpu.make_async_copy(x_hbm.at[b, pl.ds(0, CH), :],
                                  x_v.at[cur], in_sems[cur]).wait()

            def row_body(r, carry):
                for j in range(D // _LANES):
                    sl = pl.ds(j * _LANES, _LANES)
                    plsc.addupdate(x_v.at[cur, r, sl], pe_v[r, sl])
                return carry

            lax.fori_loop(0, CH, row_body, 0)

            pltpu.async_copy(x_v.at[cur],
                             out_hbm.at[b, pl.ds(s0_of(c), CH), :],
                             out_sems[cur])

            if b == B - 1 and c + 1 < n_chunks:
                pltpu.sync_copy(pe_hbm.at[pl.ds(s0_of(c + 1), CH), :], pe_v)

        last = len(steps) - 1
        for i in (last - 1, last):
            c, b = steps[i]
            pltpu.make_async_copy(x_v.at[i % 2],
                                  out_hbm.at[b, pl.ds(s0_of(c), CH), :],
                                  out_sems[i % 2]).wait()

    return k(x, pe_weight)


# speedup vs baseline: 4.6742x; 3.8519x over previous
import jax
import jax.numpy as jnp
from jax.experimental import pallas as pl


def _posemb_add_kernel(x_ref, pe_ref, o_ref):
    o_ref[...] = x_ref[...] + pe_ref[...][None, :, :]


def kernel(x, pe_weight):
    B, S, D = x.shape
    S_BLK = 512
    return pl.pallas_call(
        _posemb_add_kernel,
        grid=(S // S_BLK,),
        in_specs=[
            pl.BlockSpec((B, S_BLK, D), lambda i: (0, i, 0)),
            pl.BlockSpec((S_BLK, D), lambda i: (i, 0)),
        ],
        out_specs=pl.BlockSpec((B, S_BLK, D), lambda i: (0, i, 0)),
        out_shape=jax.ShapeDtypeStruct(x.shape, x.dtype),
    )(x, pe_weight)
